# Initial kernel scaffold; baseline (speedup 1.0000x reference)
import jax
import jax.numpy as jnp
from jax.experimental import pallas as pl


def kernel(x, edge_index, W1, att_src1, att_dst1, b1, W2, att_src2, att_dst2, b2):
    raise NotImplementedError("write your pallas kernel here")


# speedup vs baseline: 42.2291x; 42.2291x over previous
import functools

import jax
import jax.numpy as jnp
from jax import lax
from jax.experimental import pallas as pl
from jax.experimental.pallas import tpu as pltpu
from jax.experimental.pallas import tpu_sc as plsc

HEADS = 8
HID = 16
ROW1 = 144
ROW2 = 16
EB = 64
N_TILES = 32
BM = 128


def _tc1_body(x_ref, w1_ref, s1_ref, d1_ref, pack_ref, adst_ref):
    h = lax.dot_general(x_ref[...], w1_ref[...], (((1,), (1,)), ((), ())),
                        preferred_element_type=jnp.float32)
    a16 = jnp.dot(h, s1_ref[...], preferred_element_type=jnp.float32)
    adst_ref[...] = jnp.dot(h, d1_ref[...], preferred_element_type=jnp.float32)
    pack_ref[...] = jnp.concatenate([h, a16], axis=1)


def _tc2_body(p0_ref, p1_ref, b1_ref, wp_ref, wd_ref, pack2_ref, adst2_ref):
    A = p0_ref[...] + p1_ref[...]
    h = A[:, :128]
    den8 = A[:, 128:136]
    denb = jnp.concatenate(
        [lax.broadcast_in_dim(den8[:, k:k + 1], (BM, 16), (0, 1))
         for k in range(HEADS)], axis=1)
    h1 = h / denb + b1_ref[...]
    h1 = jnp.where(h1 > 0, h1, jnp.exp(h1) - 1.0)
    hp = jnp.dot(h1, wp_ref[...], preferred_element_type=jnp.float32)
    one3 = (lax.broadcasted_iota(jnp.int32, (BM, 16), 1) == 3)
    pack2_ref[...] = hp + one3.astype(jnp.float32)
    adst2_ref[...] = jnp.dot(h1, wd_ref[...], preferred_element_type=jnp.float32)


def _tc3_body(p0_ref, p1_ref, b2_ref, out_ref):
    A = p0_ref[...] + p1_ref[...]
    den = lax.broadcast_in_dim(A[:, 3:4], (BM, 16), (0, 1))
    h2 = A / den + b2_ref[...]
    aa = h2[:, 0:1]
    bb = h2[:, 1:2]
    mx = jnp.maximum(aa, bb)
    lse = mx + jnp.log(jnp.exp(aa - mx) + jnp.exp(bb - mx))
    out_ref[...] = h2 - lax.broadcast_in_dim(lse, (BM, 16), (0, 1))


def _sc1_body(nrows, epad, pack_hbm, adst_hbm, src_hbm, dst_hbm, out_hbm,
              sidx, didx, gbuf, abuf, mbuf, zbuf, acc, sem_g, sem_a):
    cid = lax.axis_index("c")
    sid = lax.axis_index("s")
    wid = cid * 16 + sid
    zero16 = jnp.zeros((16,), jnp.float32)
    lanes = lax.iota(jnp.int32, 16)
    mask8 = (lanes < HEADS).astype(jnp.float32)

    @pl.loop(0, EB)
    def _(e):
        for k in range(ROW1 // 16):
            zbuf[e, pl.ds(k * 16, 16)] = zero16
            mbuf[e, pl.ds(k * 16, 16)] = zero16

    nblk_rows = nrows // EB

    @pl.loop(0, nblk_rows)
    def _(j):
        @pl.when(lax.rem(j, 16) == sid)
        def _():
            pltpu.sync_copy(zbuf, acc.at[pl.ds(j * EB, EB)])

    plsc.subcore_barrier()

    nblk = epad // EB // N_TILES

    @pl.loop(0, nblk)
    def _(bi):
        base = (wid * nblk + bi) * EB
        pltpu.sync_copy(src_hbm.at[pl.ds(base, EB)], sidx)
        pltpu.sync_copy(dst_hbm.at[pl.ds(base, EB)], didx)
        cg = pltpu.async_copy(pack_hbm.at[sidx], gbuf, sem_g)
        ca = pltpu.async_copy(adst_hbm.at[didx], abuf, sem_a)
        cg.wait()
        ca.wait()

        @pl.loop(0, EB)
        def _(e):
            asr = gbuf[e, pl.ds(128, 16)]
            adr = abuf[e, pl.ds(0, 16)]
            al = asr + adr
            al = jnp.maximum(al, al * 0.2)
            ex = jnp.exp(al) * mask8
            mbuf[e, pl.ds(128, 16)] = ex
            for k in range(HEADS):
                sv = lax.broadcast_in_dim(ex[k], (16,), ())
                mbuf[e, pl.ds(k * 16, 16)] = gbuf[e, pl.ds(k * 16, 16)] * sv

        pltpu.sync_copy(mbuf, acc.at[didx], add=True)

    plsc.subcore_barrier()

    @pl.loop(0, nblk_rows)
    def _(j):
        @pl.when(lax.rem(j, 16) == sid)
        def _():
            pltpu.sync_copy(acc.at[pl.ds(j * EB, EB)],
                            out_hbm.at[cid, pl.ds(j * EB, EB)])


def _sc2_body(nrows, epad, pack_hbm, adst_hbm, src_hbm, dst_hbm, out_hbm,
              sidx, didx, gbuf, abuf, mbuf, zbuf, acc, sem_g, sem_a):
    cid = lax.axis_index("c")
    sid = lax.axis_index("s")
    wid = cid * 16 + sid
    zero16 = jnp.zeros((16,), jnp.float32)
    lanes = lax.iota(jnp.int32, 16)
    not2 = (lanes != 2).astype(jnp.float32)

    @pl.loop(0, EB)
    def _(e):
        zbuf[e, pl.ds(0, 16)] = zero16

    nblk_rows = nrows // EB

    @pl.loop(0, nblk_rows)
    def _(j):
        @pl.when(lax.rem(j, 16) == sid)
        def _():
            pltpu.sync_copy(zbuf, acc.at[pl.ds(j * EB, EB)])

    plsc.subcore_barrier()

    nblk = epad // EB // N_TILES

    @pl.loop(0, nblk)
    def _(bi):
        base = (wid * nblk + bi) * EB
        pltpu.sync_copy(src_hbm.at[pl.ds(base, EB)], sidx)
        pltpu.sync_copy(dst_hbm.at[pl.ds(base, EB)], didx)
        cg = pltpu.async_copy(pack_hbm.at[sidx], gbuf, sem_g)
        ca = pltpu.async_copy(adst_hbm.at[didx], abuf, sem_a)
        cg.wait()
        ca.wait()

        @pl.loop(0, EB)
        def _(e):
            v = gbuf[e, pl.ds(0, 16)]
            av = abuf[e, pl.ds(0, 16)]
            al = lax.broadcast_in_dim(v[2] + av[2], (16,), ())
            al = jnp.maximum(al, al * 0.2)
            ex = jnp.exp(al)
            mbuf[e, pl.ds(0, 16)] = ex * v * not2

        pltpu.sync_copy(mbuf, acc.at[didx], add=True)

    plsc.subcore_barrier()

    @pl.loop(0, nblk_rows)
    def _(j):
        @pl.when(lax.rem(j, 16) == sid)
        def _():
            pltpu.sync_copy(acc.at[pl.ds(j * EB, EB)],
                            out_hbm.at[cid, pl.ds(j * EB, EB)])


def _sc_layer(body, row, pack, adst, src_p, dst_p, nrows, epad):
    mesh = plsc.VectorSubcoreMesh(core_axis_name="c", subcore_axis_name="s")
    k = pl.kernel(
        functools.partial(body, nrows, epad),
        mesh=mesh,
        compiler_params=pltpu.CompilerParams(needs_layout_passes=False,
                                             use_tc_tiling_on_sc=False),
        out_type=jax.ShapeDtypeStruct((2, nrows, row), jnp.float32),
        scratch_types=[
            pltpu.VMEM((EB,), jnp.int32),
            pltpu.VMEM((EB,), jnp.int32),
            pltpu.VMEM((EB, row), jnp.float32),
            pltpu.VMEM((EB, 16), jnp.float32),
            pltpu.VMEM((EB, row), jnp.float32),
            pltpu.VMEM((EB, row), jnp.float32),
            pltpu.VMEM_SHARED((nrows, row), jnp.float32),
            pltpu.SemaphoreType.DMA,
            pltpu.SemaphoreType.DMA,
        ],
    )
    return k(pack, adst, src_p, dst_p)


def kernel(x, edge_index, W1, att_src1, att_dst1, b1, W2, att_src2, att_dst2, b2):
    n, d_in = x.shape
    e = edge_index.shape[1]
    nrows = ((n + 8) + 127) // 128 * 128
    etot = e + n
    epad = (etot + (N_TILES * EB) - 1) // (N_TILES * EB) * (N_TILES * EB)

    loop = jnp.arange(n, dtype=jnp.int32)
    src_p = jnp.concatenate(
        [edge_index[0], loop, jnp.zeros((epad - etot,), jnp.int32)])
    dst_p = jnp.concatenate(
        [edge_index[1], loop, jnp.full((epad - etot,), nrows - 1, jnp.int32)])
    x_p = jnp.pad(x, ((0, nrows - n), (0, 0)))

    eye = jnp.eye(HEADS, dtype=jnp.float32)
    S1 = (att_src1[:, :, None] * eye[:, None, :]).reshape(HEADS * HID, HEADS)
    D1 = (att_dst1[:, :, None] * eye[:, None, :]).reshape(HEADS * HID, HEADS)
    S1p = jnp.pad(S1, ((0, 0), (0, 8)))
    D1p = jnp.pad(D1, ((0, 0), (0, 8)))

    pack1, adst1 = pl.pallas_call(
        _tc1_body,
        grid=(nrows // BM,),
        in_specs=[
            pl.BlockSpec((BM, d_in), lambda i: (i, 0)),
            pl.BlockSpec((HEADS * HID, d_in), lambda i: (0, 0)),
            pl.BlockSpec((HEADS * HID, 16), lambda i: (0, 0)),
            pl.BlockSpec((HEADS * HID, 16), lambda i: (0, 0)),
        ],
        out_specs=[
            pl.BlockSpec((BM, ROW1), lambda i: (i, 0)),
            pl.BlockSpec((BM, ROW2), lambda i: (i, 0)),
        ],
        out_shape=[
            jax.ShapeDtypeStruct((nrows, ROW1), jnp.float32),
            jax.ShapeDtypeStruct((nrows, ROW2), jnp.float32),
        ],
    )(x_p, W1, S1p, D1p)

    acc1 = _sc_layer(_sc1_body, ROW1, pack1, adst1, src_p, dst_p, nrows, epad)

    v_s2 = W2.T @ att_src2[0]
    v_d2 = W2.T @ att_dst2[0]
    Wp = jnp.concatenate(
        [W2.T, v_s2[:, None], jnp.zeros((HEADS * HID, 13), jnp.float32)], axis=1)
    Wd = jnp.concatenate(
        [jnp.zeros((HEADS * HID, 2), jnp.float32), v_d2[:, None],
         jnp.zeros((HEADS * HID, 13), jnp.float32)], axis=1)
    b1r = b1.reshape(1, HEADS * HID)

    pack2, adst2 = pl.pallas_call(
        _tc2_body,
        grid=(nrows // BM,),
        in_specs=[
            pl.BlockSpec((BM, ROW1), lambda i: (i, 0)),
            pl.BlockSpec((BM, ROW1), lambda i: (i, 0)),
            pl.BlockSpec((1, HEADS * HID), lambda i: (0, 0)),
            pl.BlockSpec((HEADS * HID, 16), lambda i: (0, 0)),
            pl.BlockSpec((HEADS * HID, 16), lambda i: (0, 0)),
        ],
        out_specs=[
            pl.BlockSpec((BM, ROW2), lambda i: (i, 0)),
            pl.BlockSpec((BM, ROW2), lambda i: (i, 0)),
        ],
        out_shape=[
            jax.ShapeDtypeStruct((nrows, ROW2), jnp.float32),
            jax.ShapeDtypeStruct((nrows, ROW2), jnp.float32),
        ],
    )(acc1[0], acc1[1], b1r, Wp, Wd)

    acc2 = _sc_layer(_sc2_body, ROW2, pack2, adst2, src_p, dst_p, nrows, epad)

    b2r = jnp.pad(b2, (0, 16 - b2.shape[0])).reshape(1, 16)
    out16 = pl.pallas_call(
        _tc3_body,
        grid=(nrows // BM,),
        in_specs=[
            pl.BlockSpec((BM, ROW2), lambda i: (i, 0)),
            pl.BlockSpec((BM, ROW2), lambda i: (i, 0)),
            pl.BlockSpec((1, 16), lambda i: (0, 0)),
        ],
        out_specs=pl.BlockSpec((BM, 16), lambda i: (i, 0)),
        out_shape=jax.ShapeDtypeStruct((nrows, 16), jnp.float32),
    )(acc2[0], acc2[1], b2r)

    return out16[:n, :2]
